# Initial kernel scaffold; baseline (speedup 1.0000x reference)
import jax
import jax.numpy as jnp
from jax.experimental import pallas as pl


def kernel(h, router_in, router_w, router_b, w1, b1, w2, b2):
    raise NotImplementedError("write your pallas kernel here")


# speedup vs baseline: 2.7125x; 2.7125x over previous
import functools

import jax
import jax.numpy as jnp
from jax.experimental import pallas as pl
from jax.experimental.pallas import tpu as pltpu

N = 8192
D = 768
E = 8
K = 2
DCSI = 10
DFF = 768
BALANCE_WEIGHT = 0.5

BT = 2048
NT = N // BT


def _moe_kernel(h_ref, csi_ref, rwh_ref, rwc_ref, rb_ref,
                w1_ref, b1_ref, w2_ref, b2_ref,
                out_ref, loss_ref,
                gw_ref, sump_ref, cnt_ref):
    tb = pl.program_id(0)
    e = pl.program_id(1)

    @pl.when(e == 0)
    def _router():
        hb = h_ref[...]
        logits = (jnp.dot(hb, rwh_ref[...], preferred_element_type=jnp.float32)
                  + jnp.dot(csi_ref[...], rwc_ref[...],
                            preferred_element_type=jnp.float32)
                  + rb_ref[...])
        m = jnp.max(logits, axis=-1, keepdims=True)
        ex = jnp.exp(logits - m)
        probs = ex / jnp.sum(ex, axis=-1, keepdims=True)
        lane = jax.lax.broadcasted_iota(jnp.int32, (BT, E), 1)
        v1 = jnp.max(probs, axis=-1, keepdims=True)
        e1 = jnp.min(jnp.where(probs == v1, lane, E), axis=-1, keepdims=True)
        masked = jnp.where(lane == e1, -jnp.inf, probs)
        v2 = jnp.max(masked, axis=-1, keepdims=True)
        e2 = jnp.min(jnp.where(masked == v2, lane, E), axis=-1, keepdims=True)
        denom = v1 + v2
        g1 = v1 / denom
        g2 = v2 / denom
        sel1 = (lane == e1).astype(jnp.float32)
        sel2 = (lane == e2).astype(jnp.float32)
        gw_ref[...] = g1 * sel1 + g2 * sel2

        @pl.when(tb == 0)
        def _init():
            sump_ref[...] = jnp.zeros_like(sump_ref)
            cnt_ref[...] = jnp.zeros_like(cnt_ref)

        sump_ref[...] += jnp.sum(probs, axis=0, keepdims=True)
        cnt_ref[...] += jnp.sum(sel1 + sel2, axis=0, keepdims=True)

    hb = h_ref[...]
    hid = (jnp.dot(hb, w1_ref[0], preferred_element_type=jnp.float32)
           + b1_ref[0])
    hid = jax.nn.gelu(hid)
    y = (jnp.dot(hid, w2_ref[0], preferred_element_type=jnp.float32)
         + b2_ref[0])
    lane_e = jax.lax.broadcasted_iota(jnp.int32, (BT, E), 1)
    gwcol = jnp.sum(jnp.where(lane_e == e, gw_ref[...], 0.0),
                    axis=-1, keepdims=True)
    contrib = gwcol * y

    @pl.when(e == 0)
    def _first():
        out_ref[...] = hb + contrib

    @pl.when(e != 0)
    def _acc():
        out_ref[...] += contrib

    @pl.when(jnp.logical_and(tb == NT - 1, e == E - 1))
    def _loss():
        mean_prob = sump_ref[...] / N
        load_frac = cnt_ref[...] / (N * K)
        loss_ref[...] = (BALANCE_WEIGHT * E
                         * jnp.sum(mean_prob * load_frac,
                                   axis=-1, keepdims=True))


@jax.jit
def kernel(h, router_in, router_w, router_b, w1, b1, w2, b2):
    csi = router_in[:, -DCSI:]
    rwh = router_w[:D, :]
    rwc = router_w[D:, :]
    rb = router_b[None, :]

    out, loss = pl.pallas_call(
        _moe_kernel,
        grid=(NT, E),
        in_specs=[
            pl.BlockSpec((BT, D), lambda tb, e: (tb, 0)),
            pl.BlockSpec((BT, DCSI), lambda tb, e: (tb, 0)),
            pl.BlockSpec((D, E), lambda tb, e: (0, 0)),
            pl.BlockSpec((DCSI, E), lambda tb, e: (0, 0)),
            pl.BlockSpec((1, E), lambda tb, e: (0, 0)),
            pl.BlockSpec((1, D, DFF), lambda tb, e: (e, 0, 0)),
            pl.BlockSpec((1, 1, DFF), lambda tb, e: (e, 0, 0)),
            pl.BlockSpec((1, DFF, D), lambda tb, e: (e, 0, 0)),
            pl.BlockSpec((1, 1, D), lambda tb, e: (e, 0, 0)),
        ],
        out_specs=[
            pl.BlockSpec((BT, D), lambda tb, e: (tb, 0)),
            pl.BlockSpec((1, 1), lambda tb, e: (0, 0)),
        ],
        out_shape=[
            jax.ShapeDtypeStruct((N, D), jnp.float32),
            jax.ShapeDtypeStruct((1, 1), jnp.float32),
        ],
        scratch_shapes=[
            pltpu.VMEM((BT, E), jnp.float32),
            pltpu.VMEM((1, E), jnp.float32),
            pltpu.VMEM((1, E), jnp.float32),
        ],
    )(h, csi, rwh, rwc, rb, w1, b1[:, None, :], w2, b2[:, None, :])
    return out, loss[0, 0]
